# Initial kernel scaffold; baseline (speedup 1.0000x reference)
import functools

import jax
import jax.numpy as jnp
from jax import lax
from jax.experimental import pallas as pl
from jax.experimental.pallas import tpu as pltpu
from jax.experimental.pallas import tpu_sc as plsc

B = 16384
F = 26
D = 32
IN = F * D
H = 1024

NC = 2
NS = 16
NW = NC * NS

N = B * F
B_PER_W = N // NW
CHUNK = 1024


def _gather_sc(table, idx_flat):
    mesh = plsc.VectorSubcoreMesh(core_axis_name="c", subcore_axis_name="s")

    @functools.partial(
        pl.kernel,
        mesh=mesh,
        out_type=jax.ShapeDtypeStruct((N, D), jnp.float32),
        scratch_types=[
            pltpu.VMEM((CHUNK,), jnp.int32),
            pltpu.VMEM((CHUNK, D), jnp.float32),
            pltpu.SemaphoreType.DMA,
        ],
    )
    def gather_kernel(table_hbm, idx_hbm, out_hbm, idx_v, rows_v, sem):
        wid = lax.axis_index("s") * NC + lax.axis_index("c")
        base = wid * B_PER_W

        @pl.loop(0, B_PER_W, step=CHUNK)
        def _(off):
            pltpu.sync_copy(idx_hbm.at[pl.ds(base + off, CHUNK)], idx_v)
            pltpu.async_copy(table_hbm.at[idx_v], rows_v, sem).wait()
            pltpu.sync_copy(rows_v, out_hbm.at[pl.ds(base + off, CHUNK)])

    return gather_kernel(table, idx_flat)


def _mlp_tc(flat, W1, b1r, w2r, b2s):
    bm = 2048

    def mlp_kernel(flat_ref, w1_ref, b1_ref, w2_ref, b2_ref, out_ref):
        h = jnp.dot(flat_ref[...], w1_ref[...],
                    preferred_element_type=jnp.float32)
        h = jnp.maximum(h + b1_ref[...], 0.0)
        out_ref[...] = (jnp.sum(h * w2_ref[...], axis=1, keepdims=True)
                        + b2_ref[0, 0])

    return pl.pallas_call(
        mlp_kernel,
        grid=(B // bm,),
        in_specs=[
            pl.BlockSpec((bm, IN), lambda i: (i, 0)),
            pl.BlockSpec((IN, H), lambda i: (0, 0)),
            pl.BlockSpec((1, H), lambda i: (0, 0)),
            pl.BlockSpec((1, H), lambda i: (0, 0)),
            pl.BlockSpec((1, 1), lambda i: (0, 0), memory_space=pltpu.SMEM),
        ],
        out_specs=pl.BlockSpec((bm, 1), lambda i: (i, 0)),
        out_shape=jax.ShapeDtypeStruct((B, 1), jnp.float32),
    )(flat, W1, b1r, w2r, b2s)


def kernel(x, table, W1, b1, W2, b2):
    idx_flat = x.reshape(N).astype(jnp.int32)
    rows = _gather_sc(table, idx_flat)
    flat = rows.reshape(B, IN)
    b1r = b1.reshape(1, H)
    w2r = W2.reshape(1, H)
    b2s = b2.reshape(1, 1)
    return _mlp_tc(flat, W1, b1r, w2r, b2s)


# speedup vs baseline: 16.5270x; 16.5270x over previous
import functools

import jax
import jax.numpy as jnp
from jax import lax
from jax.experimental import pallas as pl
from jax.experimental.pallas import tpu as pltpu
from jax.experimental.pallas import tpu_sc as plsc

B = 16384
F = 26
D = 32
IN = F * D
H = 1024

NC = 2
NS = 16
NW = NC * NS

N = B * F
B_PER_W = N // NW
CHUNK = 1024


def _gather_sc(table, idx_flat):
    mesh = plsc.VectorSubcoreMesh(core_axis_name="c", subcore_axis_name="s")

    @functools.partial(
        pl.kernel,
        mesh=mesh,
        out_type=jax.ShapeDtypeStruct((N, D), jnp.float32),
        compiler_params=pltpu.CompilerParams(use_tc_tiling_on_sc=False),
        scratch_types=[
            pltpu.VMEM((CHUNK,), jnp.int32),
            pltpu.VMEM((CHUNK, D), jnp.float32),
            pltpu.SemaphoreType.DMA,
        ],
    )
    def gather_kernel(table_hbm, idx_hbm, out_hbm, idx_v, rows_v, sem):
        wid = lax.axis_index("s") * NC + lax.axis_index("c")
        base = wid * B_PER_W

        @pl.loop(0, B_PER_W, step=CHUNK)
        def _(off):
            pltpu.sync_copy(idx_hbm.at[pl.ds(base + off, CHUNK)], idx_v)
            pltpu.async_copy(table_hbm.at[idx_v], rows_v, sem).wait()
            pltpu.sync_copy(rows_v, out_hbm.at[pl.ds(base + off, CHUNK)])

    return gather_kernel(table, idx_flat)


def _mlp_tc(flat, W1, b1r, w2r, b2s):
    bm = 2048

    def mlp_kernel(flat_ref, w1_ref, b1_ref, w2_ref, b2_ref, out_ref):
        h = jnp.dot(flat_ref[...], w1_ref[...],
                    preferred_element_type=jnp.float32)
        h = jnp.maximum(h + b1_ref[...], 0.0)
        out_ref[...] = (jnp.sum(h * w2_ref[...], axis=1, keepdims=True)
                        + b2_ref[0, 0])

    return pl.pallas_call(
        mlp_kernel,
        grid=(B // bm,),
        in_specs=[
            pl.BlockSpec((bm, IN), lambda i: (i, 0)),
            pl.BlockSpec((IN, H), lambda i: (0, 0)),
            pl.BlockSpec((1, H), lambda i: (0, 0)),
            pl.BlockSpec((1, H), lambda i: (0, 0)),
            pl.BlockSpec((1, 1), lambda i: (0, 0), memory_space=pltpu.SMEM),
        ],
        out_specs=pl.BlockSpec((bm, 1), lambda i: (i, 0)),
        out_shape=jax.ShapeDtypeStruct((B, 1), jnp.float32),
    )(flat, W1, b1r, w2r, b2s)


def kernel(x, table, W1, b1, W2, b2):
    idx_flat = x.reshape(N).astype(jnp.int32)
    rows = _gather_sc(table, idx_flat)
    flat = rows.reshape(B, IN)
    b1r = b1.reshape(1, H)
    w2r = W2.reshape(1, H)
    b2s = b2.reshape(1, 1)
    return _mlp_tc(flat, W1, b1r, w2r, b2s)
